# baseline (speedup 1.0000x reference)
import jax
import jax.numpy as jnp
from jax.experimental import pallas as pl
from jax.experimental.pallas import tpu as pltpu

N = 10000
D = 128
L = 10


def _final_matmul_kernel(cat_ref, w_ref, b_ref, o_ref):
    o_ref[...] = (
        jnp.dot(cat_ref[...], w_ref[...], preferred_element_type=jnp.float32)
        + b_ref[...]
    )


def _final_linear(cat, lin_W, lin_b):
    n = cat.shape[0]
    blk = 1000
    return pl.pallas_call(
        _final_matmul_kernel,
        grid=(n // blk,),
        in_specs=[
            pl.BlockSpec((blk, cat.shape[1]), lambda i: (i, 0)),
            pl.BlockSpec((cat.shape[1], D), lambda i: (0, 0)),
            pl.BlockSpec((1, D), lambda i: (0, 0)),
        ],
        out_specs=pl.BlockSpec((blk, D), lambda i: (i, 0)),
        out_shape=jax.ShapeDtypeStruct((n, D), jnp.float32),
    )(cat, lin_W, lin_b.reshape(1, D))


def _gat_layer(x, src, dst, W, a_src, a_dst, b):
    n = x.shape[0]
    h = x @ W
    alpha_src = (h * a_src).sum(-1)
    alpha_dst = (h * a_dst).sum(-1)
    alpha = alpha_src[src] + alpha_dst[dst]
    alpha = jax.nn.leaky_relu(alpha, 0.2)
    m = jax.ops.segment_max(alpha, dst, num_segments=n)
    m = jnp.where(jnp.isfinite(m), m, 0.0)
    e = jnp.exp(alpha - m[dst])
    s = jax.ops.segment_sum(e, dst, num_segments=n)
    coef = e / (s[dst] + 1e-16)
    out = jax.ops.segment_sum(h[src] * coef[:, None], dst, num_segments=n)
    return out + b


def _pair_norm(x, eps=1e-5):
    x = x - x.mean(axis=0, keepdims=True)
    return x / jnp.sqrt(eps + (x ** 2).sum(-1).mean())


def kernel(x, edge_index, Ws, att_src, att_dst, biases, lin_W, lin_b):
    loop = jnp.arange(N, dtype=edge_index.dtype)
    src = jnp.concatenate([edge_index[0], loop])
    dst = jnp.concatenate([edge_index[1], loop])
    outs = []
    h = x
    for l in range(L):
        h = _gat_layer(h, src, dst, Ws[l], att_src[l], att_dst[l], biases[l])
        h = _pair_norm(h)
        h = jax.nn.gelu(h, approximate=False)
        outs.append(h)
    cat = jnp.concatenate(outs, axis=-1)
    return _final_linear(cat, lin_W, lin_b)


# speedup vs baseline: 15.2821x; 15.2821x over previous
import functools

import jax
import jax.numpy as jnp
from jax import lax
from jax.experimental import pallas as pl
from jax.experimental.pallas import tpu as pltpu
from jax.experimental.pallas import tpu_sc as plsc

N = 10000
D = 128
L = 10

NC = 2
NS = 16
TILES = NC * NS

N_PAD = 10240
DH = 64
G = 128
E_TOT = 320000 + N
NG = 81
C_EDGES = NG * G
E_PAD = TILES * C_EDGES
ROWS_PER_TILE = N_PAD // NS


def _sc_body(hp_hbm, asrc_hbm, adst_hbm, src_hbm, dst_hbm, c_hbm,
             outparts_hbm, sparts_hbm,
             src_v, dst_v, asrc_v, adst_v, e_v, s_v, c_v, rows_v, acc,
             gsem, ssem):
    cid = lax.axis_index("c")
    sid = lax.axis_index("s")
    tid = cid * NS + sid

    pltpu.sync_copy(src_hbm.at[tid], src_v)
    pltpu.sync_copy(dst_hbm.at[tid], dst_v)
    pltpu.sync_copy(asrc_hbm, asrc_v)
    pltpu.sync_copy(adst_hbm, adst_v)
    pltpu.sync_copy(c_hbm, c_v)
    cvec = c_v[...]

    zeros16 = jnp.zeros((16,), jnp.float32)
    base = sid * ROWS_PER_TILE

    def _zero_row(r, _):
        for j in range(DH // 16):
            rows_v[0, r, pl.ds(j * 16, 16)] = zeros16
        return 0

    def _zero_s(i, _):
        s_v[pl.ds(i * 16, 16)] = zeros16
        return 0

    lax.fori_loop(0, N_PAD // 16, _zero_s, 0)

    def _p1(g, _):
        for k in range(8):
            sl = pl.ds(k * 16, 16)
            i_s = src_v[g, sl]
            i_d = dst_v[g, sl]
            a = plsc.load_gather(asrc_v, [i_s]) + plsc.load_gather(adst_v, [i_d])
            a = jnp.maximum(a, 0.2 * a) - cvec
            e = jnp.exp(a)
            e_v[g, sl] = e
            plsc.addupdate_scatter(s_v, [i_d], e)
        return 0

    lax.fori_loop(0, NG, _p1, 0)
    pltpu.sync_copy(s_v, sparts_hbm.at[tid])

    for half in range(2):
        lax.fori_loop(0, G, _zero_row, 0)
        for k in range(ROWS_PER_TILE // G):
            pltpu.sync_copy(rows_v.at[0], acc.at[pl.ds(base + k * G, G)])
        plsc.subcore_barrier()

        def _p2(g, _):
            pltpu.async_copy(hp_hbm.at[half].at[src_v.at[g]], rows_v.at[0],
                             gsem).wait()

            def _scale(kk, _):
                e16 = e_v[g, pl.ds(kk * 16, 16)]
                for r0 in range(16):
                    cr = e16[r0]
                    r = kk * 16 + r0
                    for j in range(DH // 16):
                        sl = pl.ds(j * 16, 16)
                        rows_v[0, r, sl] = rows_v[0, r, sl] * cr
                return 0

            lax.fori_loop(0, G // 16, _scale, 0)
            pltpu.async_copy(rows_v.at[0], acc.at[dst_v.at[g]], ssem,
                             add=True).wait()
            return 0

        lax.fori_loop(0, NG, _p2, 0)
        plsc.subcore_barrier()
        pltpu.sync_copy(acc.at[pl.ds(base, ROWS_PER_TILE)],
                        outparts_hbm.at[cid, half, pl.ds(base, ROWS_PER_TILE)])


def _sc_edge_pass(hp, asrc, adst, src3, dst3, cvec):
    mesh = plsc.VectorSubcoreMesh(core_axis_name="c", subcore_axis_name="s",
                                  num_cores=NC, num_subcores=NS)
    f = pl.kernel(
        _sc_body,
        out_type=[
            jax.ShapeDtypeStruct((NC, 2, N_PAD, DH), jnp.float32),
            jax.ShapeDtypeStruct((TILES, N_PAD), jnp.float32),
        ],
        mesh=mesh,
        compiler_params=pltpu.CompilerParams(needs_layout_passes=False,
                                             use_tc_tiling_on_sc=False),
        scratch_types=[
            pltpu.VMEM((NG, G), jnp.int32),
            pltpu.VMEM((NG, G), jnp.int32),
            pltpu.VMEM((N_PAD,), jnp.float32),
            pltpu.VMEM((N_PAD,), jnp.float32),
            pltpu.VMEM((NG, G), jnp.float32),
            pltpu.VMEM((N_PAD,), jnp.float32),
            pltpu.VMEM((16,), jnp.float32),
            pltpu.VMEM((1, G, DH), jnp.float32),
            pltpu.VMEM_SHARED((N_PAD, DH), jnp.float32),
            pltpu.SemaphoreType.DMA,
            pltpu.SemaphoreType.DMA,
        ],
    )
    return f(hp, asrc, adst, src3, dst3, cvec)


def _project_body(h_ref, w_ref, as_ref, ad_ref, slab_ref, yacc_ref,
                  hp_ref, aa_ref, c_ref, yout_ref):
    h = h_ref[...]
    w = w_ref[...]
    hp = jnp.dot(h, w, preferred_element_type=jnp.float32)
    hp_ref[0] = hp[:, :DH]
    hp_ref[1] = hp[:, DH:]
    v1 = lax.dot_general(w, as_ref[...], (((1,), (1,)), ((), ())))
    v2 = lax.dot_general(w, ad_ref[...], (((1,), (1,)), ((), ())))
    a_s = jnp.dot(h, v1, preferred_element_type=jnp.float32)
    a_d = jnp.dot(h, v2, preferred_element_type=jnp.float32)
    aa_ref[...] = jnp.concatenate([a_s, a_d], axis=1)
    c = jnp.maximum(jnp.max(a_s) + jnp.max(a_d), 0.0)
    c_ref[...] = jnp.full((1, 128), c, jnp.float32)
    yout_ref[...] = yacc_ref[...] + jnp.dot(
        h, slab_ref[...], preferred_element_type=jnp.float32)


def _project(h, w, a_s, a_d, slab, yacc):
    return pl.pallas_call(
        _project_body,
        out_shape=[
            jax.ShapeDtypeStruct((2, N_PAD, DH), jnp.float32),
            jax.ShapeDtypeStruct((N_PAD, 2), jnp.float32),
            jax.ShapeDtypeStruct((1, 128), jnp.float32),
            jax.ShapeDtypeStruct((N_PAD, D), jnp.float32),
        ],
    )(h, w, a_s.reshape(1, D), a_d.reshape(1, D), slab, yacc)


def _normalize_body(p00_ref, p01_ref, p10_ref, p11_ref, sparts_ref, h_ref):
    agg = jnp.concatenate(
        [p00_ref[...] + p10_ref[...], p01_ref[...] + p11_ref[...]], axis=1)
    ones = jnp.ones((TILES, 1), jnp.float32)
    s = lax.dot_general(sparts_ref[...], ones, (((0,), (0,)), ((), ())))
    y = agg * (1.0 / (s + 1e-16))
    mask = lax.broadcasted_iota(jnp.int32, (N_PAD, 1), 0) < N
    y = jnp.where(mask, y, 0.0)
    mu = jnp.sum(y, axis=0, keepdims=True) * (1.0 / N)
    yc = jnp.where(mask, y - mu, 0.0)
    msq = jnp.sum(yc * yc) * (1.0 / N)
    x = yc * lax.rsqrt(1e-5 + msq)
    h_ref[...] = x * 0.5 * (1.0 + lax.erf(x * (2.0 ** -0.5)))


def _normalize(outparts, sparts):
    return pl.pallas_call(
        _normalize_body,
        out_shape=jax.ShapeDtypeStruct((N_PAD, D), jnp.float32),
    )(outparts[0, 0], outparts[0, 1], outparts[1, 0], outparts[1, 1], sparts)


def _final_body(h_ref, slab_ref, b_ref, yacc_ref, y_ref):
    y_ref[...] = (yacc_ref[...]
                  + jnp.dot(h_ref[...], slab_ref[...],
                            preferred_element_type=jnp.float32)
                  + b_ref[...])


def _final(h, slab, lin_b, yacc):
    return pl.pallas_call(
        _final_body,
        out_shape=jax.ShapeDtypeStruct((N_PAD, D), jnp.float32),
    )(h, slab, lin_b.reshape(1, D), yacc)


def kernel(x, edge_index, Ws, att_src, att_dst, biases, lin_W, lin_b):
    loop = jnp.arange(N, dtype=edge_index.dtype)
    pad = jnp.full((E_PAD - E_TOT,), N, dtype=edge_index.dtype)
    src = jnp.concatenate([edge_index[0], loop, pad]).astype(jnp.int32)
    dst = jnp.concatenate([edge_index[1], loop, pad]).astype(jnp.int32)
    src3 = src.reshape(TILES, NG, G)
    dst3 = dst.reshape(TILES, NG, G)

    x_pad = jnp.pad(x, ((0, N_PAD - N), (0, 0)))
    yacc = jnp.zeros((N_PAD, D), jnp.float32)
    zero_slab = jnp.zeros((D, D), jnp.float32)

    h = x_pad
    for l in range(L):
        slab = zero_slab if l == 0 else lin_W[(l - 1) * D:l * D]
        hp, aa, cmat, yacc = _project(h, Ws[l], att_src[l], att_dst[l],
                                      slab, yacc)
        asrc = aa[:, 0].reshape(N_PAD)
        adst = aa[:, 1].reshape(N_PAD)
        cvec = cmat[0, :16].reshape(16)
        outparts, sparts = _sc_edge_pass(hp, asrc, adst, src3, dst3, cvec)
        h = _normalize(outparts, sparts)

    y = _final(h, lin_W[(L - 1) * D:], lin_b, yacc)
    return y[:N]


# speedup vs baseline: 22.2146x; 1.4536x over previous
import functools

import jax
import jax.numpy as jnp
from jax import lax
from jax.experimental import pallas as pl
from jax.experimental.pallas import tpu as pltpu
from jax.experimental.pallas import tpu_sc as plsc

N = 10000
D = 128
L = 10

NC = 2
NS = 16
TILES = NC * NS

N_PAD = 10240
DH = 64
G = 128
E_TOT = 320000 + N
NG = 82
C_EDGES = NG * G
E_PAD = TILES * C_EDGES
ROWS_PER_TILE = N_PAD // NS


def _sc_body(hp_hbm, asrc_hbm, adst_hbm, src_hbm, dst_hbm, c_hbm,
             outparts_hbm, sparts_hbm,
             src_v, dst_v, e_v, c_v, acc,
             gsem0, gsem1, ssem0, ssem1):
    cid = lax.axis_index("c")
    sid = lax.axis_index("s")
    tid = cid * NS + sid

    pltpu.sync_copy(src_hbm.at[tid], src_v)
    pltpu.sync_copy(dst_hbm.at[tid], dst_v)
    pltpu.sync_copy(c_hbm, c_v)
    cvec = c_v[...]

    zeros16 = jnp.zeros((16,), jnp.float32)
    base = sid * ROWS_PER_TILE

    def _phase1(asrc_v, adst_v, s_v):
        pltpu.sync_copy(asrc_hbm, asrc_v)
        pltpu.sync_copy(adst_hbm, adst_v)

        def _zero_s(i, _):
            s_v[pl.ds(i * 16, 16)] = zeros16
            return 0

        lax.fori_loop(0, N_PAD // 16, _zero_s, 0)

        def _p1(g, _):
            for k in range(8):
                sl = pl.ds(k * 16, 16)
                i_s = src_v[g, sl]
                i_d = dst_v[g, sl]
                a = (plsc.load_gather(asrc_v, [i_s])
                     + plsc.load_gather(adst_v, [i_d]))
                a = jnp.maximum(a, 0.2 * a) - cvec
                e = jnp.exp(a)
                e_v[g, sl] = e
                plsc.addupdate_scatter(s_v, [i_d], e)
            return 0

        lax.fori_loop(0, NG, _p1, 0)
        pltpu.sync_copy(s_v, sparts_hbm.at[tid])

    pl.run_scoped(_phase1,
                  pltpu.VMEM((N_PAD,), jnp.float32),
                  pltpu.VMEM((N_PAD,), jnp.float32),
                  pltpu.VMEM((N_PAD,), jnp.float32))

    def _phase2(gbuf, sbuf):
        gsems = (gsem0, gsem1)
        ssems = (ssem0, ssem1)
        for half in range(2):
            def _zero_row(r, _):
                for j in range(DH // 16):
                    sbuf[0, r, pl.ds(j * 16, 16)] = zeros16
                return 0

            lax.fori_loop(0, G, _zero_row, 0)
            for k in range(ROWS_PER_TILE // G):
                pltpu.sync_copy(sbuf.at[0], acc.at[pl.ds(base + k * G, G)])
            plsc.subcore_barrier()

            hp_h = hp_hbm.at[half]
            pltpu.async_copy(hp_h.at[src_v.at[0]], gbuf.at[0], gsem0)
            pltpu.async_copy(hp_h.at[src_v.at[1]], gbuf.at[1], gsem1)

            def _pair(t, _):
                for p in range(2):
                    g = 2 * t + p
                    pltpu.make_async_copy(hp_h.at[src_v.at[g]], gbuf.at[p],
                                          gsems[p]).wait()

                    @pl.when(t > 0)
                    def _wait_scatter():
                        pltpu.make_async_copy(sbuf.at[p],
                                              acc.at[dst_v.at[g]],
                                              ssems[p]).wait()

                    def _scale(kk, _):
                        e16 = e_v[g, pl.ds(kk * 16, 16)]
                        for r0 in range(16):
                            cr = e16[r0]
                            r = kk * 16 + r0
                            for j in range(DH // 16):
                                sl = pl.ds(j * 16, 16)
                                sbuf[p, r, sl] = gbuf[p, r, sl] * cr
                        return 0

                    lax.fori_loop(0, G // 16, _scale, 0)

                    @pl.when(g + 2 < NG)
                    def _next_gather():
                        pltpu.async_copy(hp_h.at[src_v.at[g + 2]], gbuf.at[p],
                                         gsems[p])

                    pltpu.async_copy(sbuf.at[p], acc.at[dst_v.at[g]],
                                     ssems[p], add=True)
                return 0

            lax.fori_loop(0, NG // 2, _pair, 0)
            for p in range(2):
                pltpu.make_async_copy(sbuf.at[p], acc.at[dst_v.at[0]],
                                      ssems[p]).wait()
            plsc.subcore_barrier()
            pltpu.sync_copy(acc.at[pl.ds(base, ROWS_PER_TILE)],
                            outparts_hbm.at[cid, half,
                                            pl.ds(base, ROWS_PER_TILE)])

    pl.run_scoped(_phase2,
                  pltpu.VMEM((2, G, DH), jnp.float32),
                  pltpu.VMEM((2, G, DH), jnp.float32))


def _sc_edge_pass(hp, asrc, adst, src3, dst3, cvec):
    mesh = plsc.VectorSubcoreMesh(core_axis_name="c", subcore_axis_name="s",
                                  num_cores=NC, num_subcores=NS)
    f = pl.kernel(
        _sc_body,
        out_type=[
            jax.ShapeDtypeStruct((NC, 2, N_PAD, DH), jnp.float32),
            jax.ShapeDtypeStruct((TILES, N_PAD), jnp.float32),
        ],
        mesh=mesh,
        compiler_params=pltpu.CompilerParams(needs_layout_passes=False,
                                             use_tc_tiling_on_sc=False),
        scratch_types=[
            pltpu.VMEM((NG, G), jnp.int32),
            pltpu.VMEM((NG, G), jnp.int32),
            pltpu.VMEM((NG, G), jnp.float32),
            pltpu.VMEM((16,), jnp.float32),
            pltpu.VMEM_SHARED((N_PAD, DH), jnp.float32),
            pltpu.SemaphoreType.DMA,
            pltpu.SemaphoreType.DMA,
            pltpu.SemaphoreType.DMA,
            pltpu.SemaphoreType.DMA,
        ],
    )
    return f(hp, asrc, adst, src3, dst3, cvec)


def _project_body(h_ref, w_ref, as_ref, ad_ref, slab_ref, yacc_ref,
                  hp_ref, aa_ref, c_ref, yout_ref):
    h = h_ref[...]
    w = w_ref[...]
    hp = jnp.dot(h, w, preferred_element_type=jnp.float32)
    hp_ref[0] = hp[:, :DH]
    hp_ref[1] = hp[:, DH:]
    v1 = lax.dot_general(w, as_ref[...], (((1,), (1,)), ((), ())))
    v2 = lax.dot_general(w, ad_ref[...], (((1,), (1,)), ((), ())))
    a_s = jnp.dot(h, v1, preferred_element_type=jnp.float32)
    a_d = jnp.dot(h, v2, preferred_element_type=jnp.float32)
    aa_ref[...] = jnp.concatenate([a_s, a_d], axis=1)
    c = jnp.maximum(jnp.max(a_s) + jnp.max(a_d), 0.0)
    c_ref[...] = jnp.full((1, 128), c, jnp.float32)
    yout_ref[...] = yacc_ref[...] + jnp.dot(
        h, slab_ref[...], preferred_element_type=jnp.float32)


def _project(h, w, a_s, a_d, slab, yacc):
    return pl.pallas_call(
        _project_body,
        out_shape=[
            jax.ShapeDtypeStruct((2, N_PAD, DH), jnp.float32),
            jax.ShapeDtypeStruct((N_PAD, 2), jnp.float32),
            jax.ShapeDtypeStruct((1, 128), jnp.float32),
            jax.ShapeDtypeStruct((N_PAD, D), jnp.float32),
        ],
    )(h, w, a_s.reshape(1, D), a_d.reshape(1, D), slab, yacc)


def _normalize_body(p00_ref, p01_ref, p10_ref, p11_ref, sparts_ref, h_ref):
    agg = jnp.concatenate(
        [p00_ref[...] + p10_ref[...], p01_ref[...] + p11_ref[...]], axis=1)
    ones = jnp.ones((TILES, 1), jnp.float32)
    s = lax.dot_general(sparts_ref[...], ones, (((0,), (0,)), ((), ())))
    y = agg * (1.0 / (s + 1e-16))
    mask = lax.broadcasted_iota(jnp.int32, (N_PAD, 1), 0) < N
    y = jnp.where(mask, y, 0.0)
    mu = jnp.sum(y, axis=0, keepdims=True) * (1.0 / N)
    yc = jnp.where(mask, y - mu, 0.0)
    msq = jnp.sum(yc * yc) * (1.0 / N)
    x = yc * lax.rsqrt(1e-5 + msq)
    h_ref[...] = x * 0.5 * (1.0 + lax.erf(x * (2.0 ** -0.5)))


def _normalize(outparts, sparts):
    return pl.pallas_call(
        _normalize_body,
        out_shape=jax.ShapeDtypeStruct((N_PAD, D), jnp.float32),
    )(outparts[0, 0], outparts[0, 1], outparts[1, 0], outparts[1, 1], sparts)


def _final_body(h_ref, slab_ref, b_ref, yacc_ref, y_ref):
    y_ref[...] = (yacc_ref[...]
                  + jnp.dot(h_ref[...], slab_ref[...],
                            preferred_element_type=jnp.float32)
                  + b_ref[...])


def _final(h, slab, lin_b, yacc):
    return pl.pallas_call(
        _final_body,
        out_shape=jax.ShapeDtypeStruct((N_PAD, D), jnp.float32),
    )(h, slab, lin_b.reshape(1, D), yacc)


def kernel(x, edge_index, Ws, att_src, att_dst, biases, lin_W, lin_b):
    loop = jnp.arange(N, dtype=edge_index.dtype)
    pad = jnp.full((E_PAD - E_TOT,), N, dtype=edge_index.dtype)
    src = jnp.concatenate([edge_index[0], loop, pad]).astype(jnp.int32)
    dst = jnp.concatenate([edge_index[1], loop, pad]).astype(jnp.int32)
    src3 = src.reshape(TILES, NG, G)
    dst3 = dst.reshape(TILES, NG, G)

    x_pad = jnp.pad(x, ((0, N_PAD - N), (0, 0)))
    yacc = jnp.zeros((N_PAD, D), jnp.float32)
    zero_slab = jnp.zeros((D, D), jnp.float32)

    h = x_pad
    for l in range(L):
        slab = zero_slab if l == 0 else lin_W[(l - 1) * D:l * D]
        hp, aa, cmat, yacc = _project(h, Ws[l], att_src[l], att_dst[l],
                                      slab, yacc)
        asrc = aa[:, 0].reshape(N_PAD)
        adst = aa[:, 1].reshape(N_PAD)
        cvec = cmat[0, :16].reshape(16)
        outparts, sparts = _sc_edge_pass(hp, asrc, adst, src3, dst3, cvec)
        h = _normalize(outparts, sparts)

    y = _final(h, lin_W[(L - 1) * D:], lin_b, yacc)
    return y[:N]


# speedup vs baseline: 22.3367x; 1.0055x over previous
import functools

import jax
import jax.numpy as jnp
from jax import lax
from jax.experimental import pallas as pl
from jax.experimental.pallas import tpu as pltpu
from jax.experimental.pallas import tpu_sc as plsc

N = 10000
D = 128
L = 10

NC = 2
NS = 16
TILES = NC * NS

N_PAD = 10240
DH = 64
G = 128
E_TOT = 320000 + N
NG = 82
C_EDGES = NG * G
E_PAD = TILES * C_EDGES
ROWS_PER_TILE = N_PAD // NS


def _sc_body(hp_hbm, asrc_hbm, adst_hbm, src_hbm, dst_hbm, c_hbm,
             outparts_hbm, sparts_hbm,
             src_v, dst_v, e_v, c_v, acc,
             gsem0, gsem1, ssem0, ssem1):
    cid = lax.axis_index("c")
    sid = lax.axis_index("s")
    tid = cid * NS + sid

    pltpu.sync_copy(src_hbm.at[tid], src_v)
    pltpu.sync_copy(dst_hbm.at[tid], dst_v)
    pltpu.sync_copy(c_hbm, c_v)
    cvec = c_v[...]

    zeros16 = jnp.zeros((16,), jnp.float32)
    base = sid * ROWS_PER_TILE

    def _phase1(asrc_v, adst_v, s_v):
        pltpu.sync_copy(asrc_hbm, asrc_v)
        pltpu.sync_copy(adst_hbm, adst_v)

        @plsc.parallel_loop(0, N_PAD // 16, unroll=4)
        def _zero_s(i):
            s_v[pl.ds(i * 16, 16)] = zeros16

        def _p1(g, _):
            for k in range(8):
                sl = pl.ds(k * 16, 16)
                i_s = src_v[g, sl]
                i_d = dst_v[g, sl]
                a = (plsc.load_gather(asrc_v, [i_s])
                     + plsc.load_gather(adst_v, [i_d]))
                a = jnp.maximum(a, 0.2 * a) - cvec
                e = jnp.exp(a)
                e_v[g, sl] = e
                plsc.addupdate_scatter(s_v, [i_d], e)
            return 0

        lax.fori_loop(0, NG, _p1, 0)
        pltpu.sync_copy(s_v, sparts_hbm.at[tid])

    pl.run_scoped(_phase1,
                  pltpu.VMEM((N_PAD,), jnp.float32),
                  pltpu.VMEM((N_PAD,), jnp.float32),
                  pltpu.VMEM((N_PAD,), jnp.float32))

    def _phase2(gbuf, sbuf):
        gsems = (gsem0, gsem1)
        ssems = (ssem0, ssem1)
        for half in range(2):
            @plsc.parallel_loop(0, G, unroll=4)
            def _zero_row(r):
                for j in range(DH // 16):
                    sbuf[0, r, pl.ds(j * 16, 16)] = zeros16
            for k in range(ROWS_PER_TILE // G):
                pltpu.sync_copy(sbuf.at[0], acc.at[pl.ds(base + k * G, G)])
            plsc.subcore_barrier()

            hp_h = hp_hbm.at[half]
            pltpu.async_copy(hp_h.at[src_v.at[0]], gbuf.at[0], gsem0)
            pltpu.async_copy(hp_h.at[src_v.at[1]], gbuf.at[1], gsem1)

            def _pair(t, _):
                for p in range(2):
                    g = 2 * t + p
                    pltpu.make_async_copy(hp_h.at[src_v.at[g]], gbuf.at[p],
                                          gsems[p]).wait()

                    @pl.when(t > 0)
                    def _wait_scatter():
                        pltpu.make_async_copy(sbuf.at[p],
                                              acc.at[dst_v.at[g]],
                                              ssems[p]).wait()

                    @plsc.parallel_loop(0, G // 16, unroll=2)
                    def _scale(kk):
                        e16 = e_v[g, pl.ds(kk * 16, 16)]
                        for r0 in range(16):
                            cr = e16[r0]
                            r = kk * 16 + r0
                            for j in range(DH // 16):
                                sl = pl.ds(j * 16, 16)
                                sbuf[p, r, sl] = gbuf[p, r, sl] * cr

                    @pl.when(g + 2 < NG)
                    def _next_gather():
                        pltpu.async_copy(hp_h.at[src_v.at[g + 2]], gbuf.at[p],
                                         gsems[p])

                    pltpu.async_copy(sbuf.at[p], acc.at[dst_v.at[g]],
                                     ssems[p], add=True)
                return 0

            lax.fori_loop(0, NG // 2, _pair, 0)
            for p in range(2):
                pltpu.make_async_copy(sbuf.at[p], acc.at[dst_v.at[0]],
                                      ssems[p]).wait()
            plsc.subcore_barrier()
            pltpu.sync_copy(acc.at[pl.ds(base, ROWS_PER_TILE)],
                            outparts_hbm.at[cid, half,
                                            pl.ds(base, ROWS_PER_TILE)])

    pl.run_scoped(_phase2,
                  pltpu.VMEM((2, G, DH), jnp.float32),
                  pltpu.VMEM((2, G, DH), jnp.float32))


def _sc_edge_pass(hp, asrc, adst, src3, dst3, cvec):
    mesh = plsc.VectorSubcoreMesh(core_axis_name="c", subcore_axis_name="s",
                                  num_cores=NC, num_subcores=NS)
    f = pl.kernel(
        _sc_body,
        out_type=[
            jax.ShapeDtypeStruct((NC, 2, N_PAD, DH), jnp.float32),
            jax.ShapeDtypeStruct((TILES, N_PAD), jnp.float32),
        ],
        mesh=mesh,
        compiler_params=pltpu.CompilerParams(needs_layout_passes=False,
                                             use_tc_tiling_on_sc=False),
        scratch_types=[
            pltpu.VMEM((NG, G), jnp.int32),
            pltpu.VMEM((NG, G), jnp.int32),
            pltpu.VMEM((NG, G), jnp.float32),
            pltpu.VMEM((16,), jnp.float32),
            pltpu.VMEM_SHARED((N_PAD, DH), jnp.float32),
            pltpu.SemaphoreType.DMA,
            pltpu.SemaphoreType.DMA,
            pltpu.SemaphoreType.DMA,
            pltpu.SemaphoreType.DMA,
        ],
    )
    return f(hp, asrc, adst, src3, dst3, cvec)


def _project_body(h_ref, w_ref, as_ref, ad_ref, slab_ref, yacc_ref,
                  hp_ref, aa_ref, c_ref, yout_ref):
    h = h_ref[...]
    w = w_ref[...]
    hp = jnp.dot(h, w, preferred_element_type=jnp.float32)
    hp_ref[0] = hp[:, :DH]
    hp_ref[1] = hp[:, DH:]
    v1 = lax.dot_general(w, as_ref[...], (((1,), (1,)), ((), ())))
    v2 = lax.dot_general(w, ad_ref[...], (((1,), (1,)), ((), ())))
    a_s = jnp.dot(h, v1, preferred_element_type=jnp.float32)
    a_d = jnp.dot(h, v2, preferred_element_type=jnp.float32)
    aa_ref[...] = jnp.concatenate([a_s, a_d], axis=1)
    c = jnp.maximum(jnp.max(a_s) + jnp.max(a_d), 0.0)
    c_ref[...] = jnp.full((1, 128), c, jnp.float32)
    yout_ref[...] = yacc_ref[...] + jnp.dot(
        h, slab_ref[...], preferred_element_type=jnp.float32)


def _project(h, w, a_s, a_d, slab, yacc):
    return pl.pallas_call(
        _project_body,
        out_shape=[
            jax.ShapeDtypeStruct((2, N_PAD, DH), jnp.float32),
            jax.ShapeDtypeStruct((N_PAD, 2), jnp.float32),
            jax.ShapeDtypeStruct((1, 128), jnp.float32),
            jax.ShapeDtypeStruct((N_PAD, D), jnp.float32),
        ],
    )(h, w, a_s.reshape(1, D), a_d.reshape(1, D), slab, yacc)


def _normalize_body(p00_ref, p01_ref, p10_ref, p11_ref, sparts_ref, h_ref):
    agg = jnp.concatenate(
        [p00_ref[...] + p10_ref[...], p01_ref[...] + p11_ref[...]], axis=1)
    ones = jnp.ones((TILES, 1), jnp.float32)
    s = lax.dot_general(sparts_ref[...], ones, (((0,), (0,)), ((), ())))
    y = agg * (1.0 / (s + 1e-16))
    mask = lax.broadcasted_iota(jnp.int32, (N_PAD, 1), 0) < N
    y = jnp.where(mask, y, 0.0)
    mu = jnp.sum(y, axis=0, keepdims=True) * (1.0 / N)
    yc = jnp.where(mask, y - mu, 0.0)
    msq = jnp.sum(yc * yc) * (1.0 / N)
    x = yc * lax.rsqrt(1e-5 + msq)
    h_ref[...] = x * 0.5 * (1.0 + lax.erf(x * (2.0 ** -0.5)))


def _normalize(outparts, sparts):
    return pl.pallas_call(
        _normalize_body,
        out_shape=jax.ShapeDtypeStruct((N_PAD, D), jnp.float32),
    )(outparts[0, 0], outparts[0, 1], outparts[1, 0], outparts[1, 1], sparts)


def _final_body(h_ref, slab_ref, b_ref, yacc_ref, y_ref):
    y_ref[...] = (yacc_ref[...]
                  + jnp.dot(h_ref[...], slab_ref[...],
                            preferred_element_type=jnp.float32)
                  + b_ref[...])


def _final(h, slab, lin_b, yacc):
    return pl.pallas_call(
        _final_body,
        out_shape=jax.ShapeDtypeStruct((N_PAD, D), jnp.float32),
    )(h, slab, lin_b.reshape(1, D), yacc)


def kernel(x, edge_index, Ws, att_src, att_dst, biases, lin_W, lin_b):
    loop = jnp.arange(N, dtype=edge_index.dtype)
    pad = jnp.full((E_PAD - E_TOT,), N, dtype=edge_index.dtype)
    src = jnp.concatenate([edge_index[0], loop, pad]).astype(jnp.int32)
    dst = jnp.concatenate([edge_index[1], loop, pad]).astype(jnp.int32)
    src3 = src.reshape(TILES, NG, G)
    dst3 = dst.reshape(TILES, NG, G)

    x_pad = jnp.pad(x, ((0, N_PAD - N), (0, 0)))
    yacc = jnp.zeros((N_PAD, D), jnp.float32)
    zero_slab = jnp.zeros((D, D), jnp.float32)

    h = x_pad
    for l in range(L):
        slab = zero_slab if l == 0 else lin_W[(l - 1) * D:l * D]
        hp, aa, cmat, yacc = _project(h, Ws[l], att_src[l], att_dst[l],
                                      slab, yacc)
        asrc = aa[:, 0].reshape(N_PAD)
        adst = aa[:, 1].reshape(N_PAD)
        cvec = cmat[0, :16].reshape(16)
        outparts, sparts = _sc_edge_pass(hp, asrc, adst, src3, dst3, cvec)
        h = _normalize(outparts, sparts)

    y = _final(h, lin_W[(L - 1) * D:], lin_b, yacc)
    return y[:N]


# speedup vs baseline: 45.6438x; 2.0434x over previous
import functools

import jax
import jax.numpy as jnp
from jax import lax
from jax.experimental import pallas as pl
from jax.experimental.pallas import tpu as pltpu
from jax.experimental.pallas import tpu_sc as plsc

N = 10000
D = 128
L = 10

NC = 2
NS = 16
TILES = NC * NS

N_PAD = 10240
DH = 32
G = 128
E_TOT = 320000 + N
NG = 82
C_EDGES = NG * G
E_PAD = TILES * C_EDGES
ROWS_PER_TILE = N_PAD // NS


def _sc_body(hp_hbm, asrc_hbm, adst_hbm, src_hbm, dst_hbm, c_hbm,
             outparts_hbm, sparts_hbm,
             src_v, dst_v, e_v, c_v, acc, hps,
             gsem0, gsem1, ssem0, ssem1):
    cid = lax.axis_index("c")
    sid = lax.axis_index("s")
    tid = cid * NS + sid

    pltpu.sync_copy(src_hbm.at[tid], src_v)
    pltpu.sync_copy(dst_hbm.at[tid], dst_v)
    pltpu.sync_copy(c_hbm, c_v)
    cvec = c_v[...]

    zeros16 = jnp.zeros((16,), jnp.float32)
    base = sid * ROWS_PER_TILE

    def _phase1(asrc_v, adst_v, s_v):
        pltpu.sync_copy(asrc_hbm, asrc_v)
        pltpu.sync_copy(adst_hbm, adst_v)

        @plsc.parallel_loop(0, N_PAD // 16, unroll=4)
        def _zero_s(i):
            s_v[pl.ds(i * 16, 16)] = zeros16

        def _p1(g, _):
            for k in range(8):
                sl = pl.ds(k * 16, 16)
                i_s = src_v[g, sl]
                i_d = dst_v[g, sl]
                a = (plsc.load_gather(asrc_v, [i_s])
                     + plsc.load_gather(adst_v, [i_d]))
                a = jnp.maximum(a, 0.2 * a) - cvec
                e = jnp.exp(a)
                e_v[g, sl] = e
                plsc.addupdate_scatter(s_v, [i_d], e)
            return 0

        lax.fori_loop(0, NG, _p1, 0)
        pltpu.sync_copy(s_v, sparts_hbm.at[tid])

    pl.run_scoped(_phase1,
                  pltpu.VMEM((N_PAD,), jnp.float32),
                  pltpu.VMEM((N_PAD,), jnp.float32),
                  pltpu.VMEM((N_PAD,), jnp.float32))

    def _phase2(gbuf, sbuf):
        gsems = (gsem0, gsem1)
        ssems = (ssem0, ssem1)
        for q in range(D // DH):
            pltpu.sync_copy(hp_hbm.at[pl.ds(base, ROWS_PER_TILE),
                                      pl.ds(q * DH, DH)],
                            hps.at[pl.ds(base, ROWS_PER_TILE)])

            @plsc.parallel_loop(0, G, unroll=4)
            def _zero_row(r):
                for j in range(DH // 16):
                    sbuf[0, r, pl.ds(j * 16, 16)] = zeros16
            for k in range(ROWS_PER_TILE // G):
                pltpu.sync_copy(sbuf.at[0], acc.at[pl.ds(base + k * G, G)])
            plsc.subcore_barrier()

            pltpu.async_copy(hps.at[src_v.at[0]], gbuf.at[0], gsem0)
            pltpu.async_copy(hps.at[src_v.at[1]], gbuf.at[1], gsem1)

            def _pair(t, _):
                for p in range(2):
                    g = 2 * t + p
                    pltpu.make_async_copy(hps.at[src_v.at[g]], gbuf.at[p],
                                          gsems[p]).wait()

                    @pl.when(t > 0)
                    def _wait_scatter():
                        pltpu.make_async_copy(sbuf.at[p],
                                              acc.at[dst_v.at[g]],
                                              ssems[p]).wait()

                    @plsc.parallel_loop(0, G // 16, unroll=2)
                    def _scale(kk):
                        e16 = e_v[g, pl.ds(kk * 16, 16)]
                        for r0 in range(16):
                            cr = e16[r0]
                            r = kk * 16 + r0
                            for j in range(DH // 16):
                                sl = pl.ds(j * 16, 16)
                                sbuf[p, r, sl] = gbuf[p, r, sl] * cr

                    @pl.when(g + 2 < NG)
                    def _next_gather():
                        pltpu.async_copy(hps.at[src_v.at[g + 2]], gbuf.at[p],
                                         gsems[p])

                    pltpu.async_copy(sbuf.at[p], acc.at[dst_v.at[g]],
                                     ssems[p], add=True)
                return 0

            lax.fori_loop(0, NG // 2, _pair, 0)
            for p in range(2):
                pltpu.make_async_copy(sbuf.at[p], acc.at[dst_v.at[0]],
                                      ssems[p]).wait()
            plsc.subcore_barrier()
            pltpu.sync_copy(acc.at[pl.ds(base, ROWS_PER_TILE)],
                            outparts_hbm.at[cid, pl.ds(base, ROWS_PER_TILE),
                                            pl.ds(q * DH, DH)])

    pl.run_scoped(_phase2,
                  pltpu.VMEM((2, G, DH), jnp.float32),
                  pltpu.VMEM((2, G, DH), jnp.float32))


def _sc_edge_pass(hp, asrc, adst, src3, dst3, cvec):
    mesh = plsc.VectorSubcoreMesh(core_axis_name="c", subcore_axis_name="s",
                                  num_cores=NC, num_subcores=NS)
    f = pl.kernel(
        _sc_body,
        out_type=[
            jax.ShapeDtypeStruct((NC, N_PAD, D), jnp.float32),
            jax.ShapeDtypeStruct((TILES, N_PAD), jnp.float32),
        ],
        mesh=mesh,
        compiler_params=pltpu.CompilerParams(needs_layout_passes=False,
                                             use_tc_tiling_on_sc=False),
        scratch_types=[
            pltpu.VMEM((NG, G), jnp.int32),
            pltpu.VMEM((NG, G), jnp.int32),
            pltpu.VMEM((NG, G), jnp.float32),
            pltpu.VMEM((16,), jnp.float32),
            pltpu.VMEM_SHARED((N_PAD, DH), jnp.float32),
            pltpu.VMEM_SHARED((N_PAD, DH), jnp.float32),
            pltpu.SemaphoreType.DMA,
            pltpu.SemaphoreType.DMA,
            pltpu.SemaphoreType.DMA,
            pltpu.SemaphoreType.DMA,
        ],
    )
    return f(hp, asrc, adst, src3, dst3, cvec)


def _project_body(h_ref, w_ref, as_ref, ad_ref, slab_ref, yacc_ref,
                  hp_ref, aa_ref, c_ref, yout_ref):
    h = h_ref[...]
    w = w_ref[...]
    hp_ref[...] = jnp.dot(h, w, preferred_element_type=jnp.float32)
    v1 = lax.dot_general(w, as_ref[...], (((1,), (1,)), ((), ())))
    v2 = lax.dot_general(w, ad_ref[...], (((1,), (1,)), ((), ())))
    a_s = jnp.dot(h, v1, preferred_element_type=jnp.float32)
    a_d = jnp.dot(h, v2, preferred_element_type=jnp.float32)
    aa_ref[...] = jnp.concatenate([a_s, a_d], axis=1)
    c = jnp.maximum(jnp.max(a_s) + jnp.max(a_d), 0.0)
    c_ref[...] = jnp.full((1, 128), c, jnp.float32)
    yout_ref[...] = yacc_ref[...] + jnp.dot(
        h, slab_ref[...], preferred_element_type=jnp.float32)


def _project(h, w, a_s, a_d, slab, yacc):
    return pl.pallas_call(
        _project_body,
        out_shape=[
            jax.ShapeDtypeStruct((N_PAD, D), jnp.float32),
            jax.ShapeDtypeStruct((N_PAD, 2), jnp.float32),
            jax.ShapeDtypeStruct((1, 128), jnp.float32),
            jax.ShapeDtypeStruct((N_PAD, D), jnp.float32),
        ],
    )(h, w, a_s.reshape(1, D), a_d.reshape(1, D), slab, yacc)


def _normalize_body(p0_ref, p1_ref, sparts_ref, h_ref):
    agg = p0_ref[...] + p1_ref[...]
    ones = jnp.ones((TILES, 1), jnp.float32)
    s = lax.dot_general(sparts_ref[...], ones, (((0,), (0,)), ((), ())))
    y = agg * (1.0 / (s + 1e-16))
    mask = lax.broadcasted_iota(jnp.int32, (N_PAD, 1), 0) < N
    y = jnp.where(mask, y, 0.0)
    mu = jnp.sum(y, axis=0, keepdims=True) * (1.0 / N)
    yc = jnp.where(mask, y - mu, 0.0)
    msq = jnp.sum(yc * yc) * (1.0 / N)
    x = yc * lax.rsqrt(1e-5 + msq)
    h_ref[...] = x * 0.5 * (1.0 + lax.erf(x * (2.0 ** -0.5)))


def _normalize(outparts, sparts):
    return pl.pallas_call(
        _normalize_body,
        out_shape=jax.ShapeDtypeStruct((N_PAD, D), jnp.float32),
    )(outparts[0], outparts[1], sparts)


def _final_body(h_ref, slab_ref, b_ref, yacc_ref, y_ref):
    y_ref[...] = (yacc_ref[...]
                  + jnp.dot(h_ref[...], slab_ref[...],
                            preferred_element_type=jnp.float32)
                  + b_ref[...])


def _final(h, slab, lin_b, yacc):
    return pl.pallas_call(
        _final_body,
        out_shape=jax.ShapeDtypeStruct((N_PAD, D), jnp.float32),
    )(h, slab, lin_b.reshape(1, D), yacc)


def kernel(x, edge_index, Ws, att_src, att_dst, biases, lin_W, lin_b):
    loop = jnp.arange(N, dtype=edge_index.dtype)
    pad = jnp.full((E_PAD - E_TOT,), N, dtype=edge_index.dtype)
    src = jnp.concatenate([edge_index[0], loop, pad]).astype(jnp.int32)
    dst = jnp.concatenate([edge_index[1], loop, pad]).astype(jnp.int32)
    src3 = src.reshape(TILES, NG, G)
    dst3 = dst.reshape(TILES, NG, G)

    x_pad = jnp.pad(x, ((0, N_PAD - N), (0, 0)))
    yacc = jnp.zeros((N_PAD, D), jnp.float32)
    zero_slab = jnp.zeros((D, D), jnp.float32)

    h = x_pad
    for l in range(L):
        slab = zero_slab if l == 0 else lin_W[(l - 1) * D:l * D]
        hp, aa, cmat, yacc = _project(h, Ws[l], att_src[l], att_dst[l],
                                      slab, yacc)
        asrc = aa[:, 0].reshape(N_PAD)
        adst = aa[:, 1].reshape(N_PAD)
        cvec = cmat[0, :16].reshape(16)
        outparts, sparts = _sc_edge_pass(hp, asrc, adst, src3, dst3, cvec)
        h = _normalize(outparts, sparts)

    y = _final(h, lin_W[(L - 1) * D:], lin_b, yacc)
    return y[:N]


# speedup vs baseline: 46.5702x; 1.0203x over previous
import functools

import jax
import jax.numpy as jnp
from jax import lax
from jax.experimental import pallas as pl
from jax.experimental.pallas import tpu as pltpu
from jax.experimental.pallas import tpu_sc as plsc

N = 10000
D = 128
L = 10

NC = 2
NS = 16
TILES = NC * NS

N_PAD = 10240
DH = 32
G = 128
E_TOT = 320000 + N
NG = 82
C_EDGES = NG * G
E_PAD = TILES * C_EDGES
ROWS_PER_TILE = N_PAD // NS


def _sc_body(hp_hbm, asrc_hbm, adst_hbm, src_hbm, dst_hbm, c_hbm,
             outparts_hbm, sparts_hbm,
             src_v, dst_v, e_v, c_v, acc, hps,
             gsem0, gsem1, ssem0, ssem1):
    cid = lax.axis_index("c")
    sid = lax.axis_index("s")
    tid = cid * NS + sid

    pltpu.sync_copy(src_hbm.at[tid], src_v)
    pltpu.sync_copy(dst_hbm.at[tid], dst_v)
    pltpu.sync_copy(c_hbm, c_v)
    cvec = c_v[...]

    zeros16 = jnp.zeros((16,), jnp.float32)
    base = sid * ROWS_PER_TILE

    def _phase1(asrc_v, adst_v, s_v):
        pltpu.sync_copy(asrc_hbm, asrc_v)
        pltpu.sync_copy(adst_hbm, adst_v)

        @plsc.parallel_loop(0, N_PAD // 16, unroll=4)
        def _zero_s(i):
            s_v[pl.ds(i * 16, 16)] = zeros16

        def _p1(g, _):
            for k in range(8):
                sl = pl.ds(k * 16, 16)
                i_s = src_v[g, sl]
                i_d = dst_v[g, sl]
                a = (plsc.load_gather(asrc_v, [i_s])
                     + plsc.load_gather(adst_v, [i_d]))
                a = jnp.maximum(a, 0.2 * a) - cvec
                e = jnp.exp(a)
                e_v[g, sl] = e
                plsc.addupdate_scatter(s_v, [i_d], e)
            return 0

        lax.fori_loop(0, NG, _p1, 0)
        pltpu.sync_copy(s_v, sparts_hbm.at[tid])

    pl.run_scoped(_phase1,
                  pltpu.VMEM((N_PAD,), jnp.float32),
                  pltpu.VMEM((N_PAD,), jnp.float32),
                  pltpu.VMEM((N_PAD,), jnp.float32))

    def _phase2(gbuf, sbuf):
        gsems = (gsem0, gsem1)
        ssems = (ssem0, ssem1)
        for q in range(D // DH):
            pltpu.sync_copy(hp_hbm.at[pl.ds(base, ROWS_PER_TILE),
                                      pl.ds(q * DH, DH)],
                            hps.at[pl.ds(base, ROWS_PER_TILE)])

            @plsc.parallel_loop(0, G, unroll=4)
            def _zero_row(r):
                for j in range(DH // 16):
                    sbuf[0, r, pl.ds(j * 16, 16)] = zeros16
            for k in range(ROWS_PER_TILE // G):
                pltpu.sync_copy(sbuf.at[0], acc.at[pl.ds(base + k * G, G)])
            plsc.subcore_barrier()

            pltpu.async_copy(hps.at[src_v.at[0]], gbuf.at[0], gsem0)
            pltpu.async_copy(hps.at[src_v.at[1]], gbuf.at[1], gsem1)

            def _pair(t, _):
                for p in range(2):
                    g = 2 * t + p
                    pltpu.make_async_copy(hps.at[src_v.at[g]], gbuf.at[p],
                                          gsems[p]).wait()

                    @pl.when(t > 0)
                    def _wait_scatter():
                        pltpu.make_async_copy(sbuf.at[p],
                                              acc.at[dst_v.at[g]],
                                              ssems[p]).wait()

                    @plsc.parallel_loop(0, G // 16, unroll=2)
                    def _scale(kk):
                        e16 = e_v[g, pl.ds(kk * 16, 16)]
                        for r0 in range(16):
                            cr = e16[r0]
                            r = kk * 16 + r0
                            for j in range(DH // 16):
                                sl = pl.ds(j * 16, 16)
                                sbuf[p, r, sl] = gbuf[p, r, sl] * cr

                    @pl.when(g + 2 < NG)
                    def _next_gather():
                        pltpu.async_copy(hps.at[src_v.at[g + 2]], gbuf.at[p],
                                         gsems[p])

                    pltpu.async_copy(sbuf.at[p], acc.at[dst_v.at[g]],
                                     ssems[p], add=True)
                return 0

            lax.fori_loop(0, NG // 2, _pair, 0)
            for p in range(2):
                pltpu.make_async_copy(sbuf.at[p], acc.at[dst_v.at[0]],
                                      ssems[p]).wait()
            plsc.subcore_barrier()
            pltpu.sync_copy(acc.at[pl.ds(base, ROWS_PER_TILE)],
                            outparts_hbm.at[cid, pl.ds(base, ROWS_PER_TILE),
                                            pl.ds(q * DH, DH)])

    pl.run_scoped(_phase2,
                  pltpu.VMEM((2, G, DH), jnp.float32),
                  pltpu.VMEM((2, G, DH), jnp.float32))


def _sc_edge_pass(hp, asrc, adst, src3, dst3, cvec):
    mesh = plsc.VectorSubcoreMesh(core_axis_name="c", subcore_axis_name="s",
                                  num_cores=NC, num_subcores=NS)
    f = pl.kernel(
        _sc_body,
        out_type=[
            jax.ShapeDtypeStruct((NC, N_PAD, D), jnp.float32),
            jax.ShapeDtypeStruct((TILES, N_PAD), jnp.float32),
        ],
        mesh=mesh,
        compiler_params=pltpu.CompilerParams(needs_layout_passes=False,
                                             use_tc_tiling_on_sc=False),
        scratch_types=[
            pltpu.VMEM((NG, G), jnp.int32),
            pltpu.VMEM((NG, G), jnp.int32),
            pltpu.VMEM((NG, G), jnp.float32),
            pltpu.VMEM((16,), jnp.float32),
            pltpu.VMEM_SHARED((N_PAD, DH), jnp.float32),
            pltpu.VMEM_SHARED((N_PAD, DH), jnp.float32),
            pltpu.SemaphoreType.DMA,
            pltpu.SemaphoreType.DMA,
            pltpu.SemaphoreType.DMA,
            pltpu.SemaphoreType.DMA,
        ],
    )
    return f(hp, asrc, adst, src3, dst3, cvec)


def _project_body(h_ref, w_ref, as_ref, ad_ref, slab_ref, yacc_ref,
                  hp_ref, aa_ref, c_ref, yout_ref):
    h = h_ref[...]
    w = w_ref[...]
    hp_ref[...] = jnp.dot(h, w, preferred_element_type=jnp.float32)
    v1 = lax.dot_general(w, as_ref[...], (((1,), (1,)), ((), ())))
    v2 = lax.dot_general(w, ad_ref[...], (((1,), (1,)), ((), ())))
    a_s = jnp.dot(h, v1, preferred_element_type=jnp.float32)
    a_d = jnp.dot(h, v2, preferred_element_type=jnp.float32)
    aa_ref[...] = jnp.concatenate([a_s, a_d], axis=1)
    c = jnp.maximum(jnp.max(a_s) + jnp.max(a_d), 0.0)
    c_ref[...] = jnp.full((1, 128), c, jnp.float32)
    yout_ref[...] = yacc_ref[...] + jnp.dot(
        h, slab_ref[...], preferred_element_type=jnp.float32)


def _project(h, w, a_s, a_d, slab, yacc):
    return pl.pallas_call(
        _project_body,
        out_shape=[
            jax.ShapeDtypeStruct((N_PAD, D), jnp.float32),
            jax.ShapeDtypeStruct((N_PAD, 2), jnp.float32),
            jax.ShapeDtypeStruct((1, 128), jnp.float32),
            jax.ShapeDtypeStruct((N_PAD, D), jnp.float32),
        ],
    )(h, w, a_s.reshape(1, D), a_d.reshape(1, D), slab, yacc)


def _norm_h(p0, p1, sparts):
    agg = p0 + p1
    ones = jnp.ones((TILES, 1), jnp.float32)
    s = lax.dot_general(sparts, ones, (((0,), (0,)), ((), ())))
    y = agg * (1.0 / (s + 1e-16))
    mask = lax.broadcasted_iota(jnp.int32, (N_PAD, 1), 0) < N
    y = jnp.where(mask, y, 0.0)
    mu = jnp.sum(y, axis=0, keepdims=True) * (1.0 / N)
    yc = jnp.where(mask, y - mu, 0.0)
    msq = jnp.sum(yc * yc) * (1.0 / N)
    x = yc * lax.rsqrt(1e-5 + msq)
    return x * 0.5 * (1.0 + lax.erf(x * (2.0 ** -0.5)))


def _mid_body(p0_ref, p1_ref, sparts_ref, w_ref, as_ref, ad_ref, slab_ref,
              yacc_ref, hp_ref, aa_ref, c_ref, yout_ref):
    h = _norm_h(p0_ref[...], p1_ref[...], sparts_ref[...])
    w = w_ref[...]
    hp_ref[...] = jnp.dot(h, w, preferred_element_type=jnp.float32)
    v1 = lax.dot_general(w, as_ref[...], (((1,), (1,)), ((), ())))
    v2 = lax.dot_general(w, ad_ref[...], (((1,), (1,)), ((), ())))
    a_s = jnp.dot(h, v1, preferred_element_type=jnp.float32)
    a_d = jnp.dot(h, v2, preferred_element_type=jnp.float32)
    aa_ref[...] = jnp.concatenate([a_s, a_d], axis=1)
    c = jnp.maximum(jnp.max(a_s) + jnp.max(a_d), 0.0)
    c_ref[...] = jnp.full((1, 128), c, jnp.float32)
    yout_ref[...] = yacc_ref[...] + jnp.dot(
        h, slab_ref[...], preferred_element_type=jnp.float32)


def _mid(outparts, sparts, w, a_s, a_d, slab, yacc):
    return pl.pallas_call(
        _mid_body,
        out_shape=[
            jax.ShapeDtypeStruct((N_PAD, D), jnp.float32),
            jax.ShapeDtypeStruct((N_PAD, 2), jnp.float32),
            jax.ShapeDtypeStruct((1, 128), jnp.float32),
            jax.ShapeDtypeStruct((N_PAD, D), jnp.float32),
        ],
    )(outparts[0], outparts[1], sparts, w, a_s.reshape(1, D),
      a_d.reshape(1, D), slab, yacc)


def _final_body(p0_ref, p1_ref, sparts_ref, slab_ref, b_ref, yacc_ref, y_ref):
    h = _norm_h(p0_ref[...], p1_ref[...], sparts_ref[...])
    y_ref[...] = (yacc_ref[...]
                  + jnp.dot(h, slab_ref[...],
                            preferred_element_type=jnp.float32)
                  + b_ref[...])


def _final(outparts, sparts, slab, lin_b, yacc):
    return pl.pallas_call(
        _final_body,
        out_shape=jax.ShapeDtypeStruct((N_PAD, D), jnp.float32),
    )(outparts[0], outparts[1], sparts, slab, lin_b.reshape(1, D), yacc)


def kernel(x, edge_index, Ws, att_src, att_dst, biases, lin_W, lin_b):
    loop = jnp.arange(N, dtype=edge_index.dtype)
    pad = jnp.full((E_PAD - E_TOT,), N, dtype=edge_index.dtype)
    src = jnp.concatenate([edge_index[0], loop, pad]).astype(jnp.int32)
    dst = jnp.concatenate([edge_index[1], loop, pad]).astype(jnp.int32)
    src3 = src.reshape(TILES, NG, G)
    dst3 = dst.reshape(TILES, NG, G)

    x_pad = jnp.pad(x, ((0, N_PAD - N), (0, 0)))
    yacc = jnp.zeros((N_PAD, D), jnp.float32)
    zero_slab = jnp.zeros((D, D), jnp.float32)

    hp, aa, cmat, yacc = _project(x_pad, Ws[0], att_src[0], att_dst[0],
                                  zero_slab, yacc)
    for l in range(L):
        asrc = aa[:, 0].reshape(N_PAD)
        adst = aa[:, 1].reshape(N_PAD)
        cvec = cmat[0, :16].reshape(16)
        outparts, sparts = _sc_edge_pass(hp, asrc, adst, src3, dst3, cvec)
        if l < L - 1:
            hp, aa, cmat, yacc = _mid(outparts, sparts, Ws[l + 1],
                                      att_src[l + 1], att_dst[l + 1],
                                      lin_W[l * D:(l + 1) * D], yacc)
    y = _final(outparts, sparts, lin_W[(L - 1) * D:], lin_b, yacc)
    return y[:N]


# speedup vs baseline: 47.4710x; 1.0193x over previous
import functools

import jax
import jax.numpy as jnp
from jax import lax
from jax.experimental import pallas as pl
from jax.experimental.pallas import tpu as pltpu
from jax.experimental.pallas import tpu_sc as plsc

N = 10000
D = 128
L = 10

NC = 2
NS = 16
TILES = NC * NS

N_PAD = 10240
DH = 32
G = 128
E_TOT = 320000 + N
NG = 82
C_EDGES = NG * G
E_PAD = TILES * C_EDGES
ROWS_PER_TILE = N_PAD // NS


def _sc_body(hp_hbm, asrc_hbm, adst_hbm, src_hbm, dst_hbm, c_hbm,
             outparts_hbm, sparts_hbm,
             src_v, dst_v, e_v, c_v, zbuf, acc, hps,
             gsem0, gsem1, ssem0, ssem1):
    cid = lax.axis_index("c")
    sid = lax.axis_index("s")
    tid = cid * NS + sid

    pltpu.sync_copy(src_hbm.at[tid], src_v)
    pltpu.sync_copy(dst_hbm.at[tid], dst_v)
    pltpu.sync_copy(c_hbm, c_v)
    cvec = c_v[...]

    zeros16 = jnp.zeros((16,), jnp.float32)
    base = sid * ROWS_PER_TILE

    @plsc.parallel_loop(0, G, unroll=4)
    def _zero_zbuf(r):
        for j in range(DH // 16):
            zbuf[r, pl.ds(j * 16, 16)] = zeros16

    def _phase1(asrc_v, adst_v, s_v):
        pltpu.sync_copy(asrc_hbm, asrc_v)
        pltpu.sync_copy(adst_hbm, adst_v)

        @plsc.parallel_loop(0, N_PAD // 16, unroll=4)
        def _zero_s(i):
            s_v[pl.ds(i * 16, 16)] = zeros16

        def _p1(g, _):
            for k in range(8):
                sl = pl.ds(k * 16, 16)
                i_s = src_v[g, sl]
                i_d = dst_v[g, sl]
                a = (plsc.load_gather(asrc_v, [i_s])
                     + plsc.load_gather(adst_v, [i_d]))
                a = jnp.maximum(a, 0.2 * a) - cvec
                e = jnp.exp(a)
                e_v[g, sl] = e
                plsc.addupdate_scatter(s_v, [i_d], e)
            return 0

        lax.fori_loop(0, NG, _p1, 0)
        pltpu.sync_copy(s_v, sparts_hbm.at[tid])

    pl.run_scoped(_phase1,
                  pltpu.VMEM((N_PAD,), jnp.float32),
                  pltpu.VMEM((N_PAD,), jnp.float32),
                  pltpu.VMEM((N_PAD,), jnp.float32))

    def _phase2(gbuf, sbuf):
        gsems = (gsem0, gsem1)
        ssems = (ssem0, ssem1)
        for q in range(D // DH):
            stage = pltpu.make_async_copy(
                hp_hbm.at[pl.ds(base, ROWS_PER_TILE), pl.ds(q * DH, DH)],
                hps.at[pl.ds(base, ROWS_PER_TILE)], gsem0)
            stage.start()
            zs = [pltpu.make_async_copy(zbuf,
                                        acc.at[pl.ds(base + k * G, G)],
                                        ssem0)
                  for k in range(ROWS_PER_TILE // G)]
            for z in zs:
                z.start()
            stage.wait()
            for z in zs:
                z.wait()
            plsc.subcore_barrier()

            pltpu.async_copy(hps.at[src_v.at[0]], gbuf.at[0], gsem0)
            pltpu.async_copy(hps.at[src_v.at[1]], gbuf.at[1], gsem1)

            def _pair(t, _):
                for p in range(2):
                    g = 2 * t + p
                    pltpu.make_async_copy(hps.at[src_v.at[g]], gbuf.at[p],
                                          gsems[p]).wait()

                    @pl.when(t > 0)
                    def _wait_scatter():
                        pltpu.make_async_copy(sbuf.at[p],
                                              acc.at[dst_v.at[g]],
                                              ssems[p]).wait()

                    @plsc.parallel_loop(0, G // 16, unroll=2)
                    def _scale(kk):
                        e16 = e_v[g, pl.ds(kk * 16, 16)]
                        for r0 in range(16):
                            cr = e16[r0]
                            r = kk * 16 + r0
                            for j in range(DH // 16):
                                sl = pl.ds(j * 16, 16)
                                sbuf[p, r, sl] = gbuf[p, r, sl] * cr

                    @pl.when(g + 2 < NG)
                    def _next_gather():
                        pltpu.async_copy(hps.at[src_v.at[g + 2]], gbuf.at[p],
                                         gsems[p])

                    pltpu.async_copy(sbuf.at[p], acc.at[dst_v.at[g]],
                                     ssems[p], add=True)
                return 0

            lax.fori_loop(0, NG // 2, _pair, 0)
            for p in range(2):
                pltpu.make_async_copy(sbuf.at[p], acc.at[dst_v.at[0]],
                                      ssems[p]).wait()
            plsc.subcore_barrier()
            pltpu.sync_copy(acc.at[pl.ds(base, ROWS_PER_TILE)],
                            outparts_hbm.at[cid, pl.ds(base, ROWS_PER_TILE),
                                            pl.ds(q * DH, DH)])

    pl.run_scoped(_phase2,
                  pltpu.VMEM((2, G, DH), jnp.float32),
                  pltpu.VMEM((2, G, DH), jnp.float32))


def _sc_edge_pass(hp, asrc, adst, src3, dst3, cvec):
    mesh = plsc.VectorSubcoreMesh(core_axis_name="c", subcore_axis_name="s",
                                  num_cores=NC, num_subcores=NS)
    f = pl.kernel(
        _sc_body,
        out_type=[
            jax.ShapeDtypeStruct((NC, N_PAD, D), jnp.float32),
            jax.ShapeDtypeStruct((TILES, N_PAD), jnp.float32),
        ],
        mesh=mesh,
        compiler_params=pltpu.CompilerParams(needs_layout_passes=False,
                                             use_tc_tiling_on_sc=False),
        scratch_types=[
            pltpu.VMEM((NG, G), jnp.int32),
            pltpu.VMEM((NG, G), jnp.int32),
            pltpu.VMEM((NG, G), jnp.float32),
            pltpu.VMEM((16,), jnp.float32),
            pltpu.VMEM((G, DH), jnp.float32),
            pltpu.VMEM_SHARED((N_PAD, DH), jnp.float32),
            pltpu.VMEM_SHARED((N_PAD, DH), jnp.float32),
            pltpu.SemaphoreType.DMA,
            pltpu.SemaphoreType.DMA,
            pltpu.SemaphoreType.DMA,
            pltpu.SemaphoreType.DMA,
        ],
    )
    return f(hp, asrc, adst, src3, dst3, cvec)


def _project_body(h_ref, w_ref, as_ref, ad_ref, slab_ref, yacc_ref,
                  hp_ref, aa_ref, c_ref, yout_ref):
    h = h_ref[...]
    w = w_ref[...]
    hp_ref[...] = jnp.dot(h, w, preferred_element_type=jnp.float32)
    v1 = lax.dot_general(w, as_ref[...], (((1,), (1,)), ((), ())))
    v2 = lax.dot_general(w, ad_ref[...], (((1,), (1,)), ((), ())))
    a_s = jnp.dot(h, v1, preferred_element_type=jnp.float32)
    a_d = jnp.dot(h, v2, preferred_element_type=jnp.float32)
    aa_ref[...] = jnp.concatenate([a_s, a_d], axis=1)
    c = jnp.maximum(jnp.max(a_s) + jnp.max(a_d), 0.0)
    c_ref[...] = jnp.full((1, 128), c, jnp.float32)
    yout_ref[...] = yacc_ref[...] + jnp.dot(
        h, slab_ref[...], preferred_element_type=jnp.float32)


def _project(h, w, a_s, a_d, slab, yacc):
    return pl.pallas_call(
        _project_body,
        out_shape=[
            jax.ShapeDtypeStruct((N_PAD, D), jnp.float32),
            jax.ShapeDtypeStruct((N_PAD, 2), jnp.float32),
            jax.ShapeDtypeStruct((1, 128), jnp.float32),
            jax.ShapeDtypeStruct((N_PAD, D), jnp.float32),
        ],
    )(h, w, a_s.reshape(1, D), a_d.reshape(1, D), slab, yacc)


def _norm_h(p0, p1, sparts):
    agg = p0 + p1
    ones = jnp.ones((TILES, 1), jnp.float32)
    s = lax.dot_general(sparts, ones, (((0,), (0,)), ((), ())))
    y = agg * (1.0 / (s + 1e-16))
    mask = lax.broadcasted_iota(jnp.int32, (N_PAD, 1), 0) < N
    y = jnp.where(mask, y, 0.0)
    mu = jnp.sum(y, axis=0, keepdims=True) * (1.0 / N)
    yc = jnp.where(mask, y - mu, 0.0)
    msq = jnp.sum(yc * yc) * (1.0 / N)
    x = yc * lax.rsqrt(1e-5 + msq)
    return x * 0.5 * (1.0 + lax.erf(x * (2.0 ** -0.5)))


def _mid_body(p0_ref, p1_ref, sparts_ref, w_ref, as_ref, ad_ref, slab_ref,
              yacc_ref, hp_ref, aa_ref, c_ref, yout_ref):
    h = _norm_h(p0_ref[...], p1_ref[...], sparts_ref[...])
    w = w_ref[...]
    hp_ref[...] = jnp.dot(h, w, preferred_element_type=jnp.float32)
    v1 = lax.dot_general(w, as_ref[...], (((1,), (1,)), ((), ())))
    v2 = lax.dot_general(w, ad_ref[...], (((1,), (1,)), ((), ())))
    a_s = jnp.dot(h, v1, preferred_element_type=jnp.float32)
    a_d = jnp.dot(h, v2, preferred_element_type=jnp.float32)
    aa_ref[...] = jnp.concatenate([a_s, a_d], axis=1)
    c = jnp.maximum(jnp.max(a_s) + jnp.max(a_d), 0.0)
    c_ref[...] = jnp.full((1, 128), c, jnp.float32)
    yout_ref[...] = yacc_ref[...] + jnp.dot(
        h, slab_ref[...], preferred_element_type=jnp.float32)


def _mid(outparts, sparts, w, a_s, a_d, slab, yacc):
    return pl.pallas_call(
        _mid_body,
        out_shape=[
            jax.ShapeDtypeStruct((N_PAD, D), jnp.float32),
            jax.ShapeDtypeStruct((N_PAD, 2), jnp.float32),
            jax.ShapeDtypeStruct((1, 128), jnp.float32),
            jax.ShapeDtypeStruct((N_PAD, D), jnp.float32),
        ],
    )(outparts[0], outparts[1], sparts, w, a_s.reshape(1, D),
      a_d.reshape(1, D), slab, yacc)


def _final_body(p0_ref, p1_ref, sparts_ref, slab_ref, b_ref, yacc_ref, y_ref):
    h = _norm_h(p0_ref[...], p1_ref[...], sparts_ref[...])
    y_ref[...] = (yacc_ref[...]
                  + jnp.dot(h, slab_ref[...],
                            preferred_element_type=jnp.float32)
                  + b_ref[...])


def _final(outparts, sparts, slab, lin_b, yacc):
    return pl.pallas_call(
        _final_body,
        out_shape=jax.ShapeDtypeStruct((N_PAD, D), jnp.float32),
    )(outparts[0], outparts[1], sparts, slab, lin_b.reshape(1, D), yacc)


def kernel(x, edge_index, Ws, att_src, att_dst, biases, lin_W, lin_b):
    loop = jnp.arange(N, dtype=edge_index.dtype)
    pad = jnp.full((E_PAD - E_TOT,), N, dtype=edge_index.dtype)
    src = jnp.concatenate([edge_index[0], loop, pad]).astype(jnp.int32)
    dst = jnp.concatenate([edge_index[1], loop, pad]).astype(jnp.int32)
    src3 = src.reshape(TILES, NG, G)
    dst3 = dst.reshape(TILES, NG, G)

    x_pad = jnp.pad(x, ((0, N_PAD - N), (0, 0)))
    yacc = jnp.zeros((N_PAD, D), jnp.float32)
    zero_slab = jnp.zeros((D, D), jnp.float32)

    hp, aa, cmat, yacc = _project(x_pad, Ws[0], att_src[0], att_dst[0],
                                  zero_slab, yacc)
    for l in range(L):
        asrc = aa[:, 0].reshape(N_PAD)
        adst = aa[:, 1].reshape(N_PAD)
        cvec = cmat[0, :16].reshape(16)
        outparts, sparts = _sc_edge_pass(hp, asrc, adst, src3, dst3, cvec)
        if l < L - 1:
            hp, aa, cmat, yacc = _mid(outparts, sparts, Ws[l + 1],
                                      att_src[l + 1], att_dst[l + 1],
                                      lin_W[l * D:(l + 1) * D], yacc)
    y = _final(outparts, sparts, lin_W[(L - 1) * D:], lin_b, yacc)
    return y[:N]


# speedup vs baseline: 49.8931x; 1.0510x over previous
import functools

import jax
import jax.numpy as jnp
from jax import lax
from jax.experimental import pallas as pl
from jax.experimental.pallas import tpu as pltpu
from jax.experimental.pallas import tpu_sc as plsc

N = 10000
D = 128
L = 10

NC = 2
NS = 16
TILES = NC * NS

N_PAD = 10240
DH = 32
G = 128
E_TOT = 320000 + N
NG = 82
C_EDGES = NG * G
E_PAD = TILES * C_EDGES
ROWS_PER_TILE = N_PAD // NS


def _sc_body(hp_hbm, asrc_hbm, adst_hbm, src_hbm, dst_hbm, c_hbm,
             outparts_hbm, sparts_hbm,
             src_v, dst_v, e_v, c_v, zbuf, acc, hps,
             gsem0, gsem1, ssem0, ssem1):
    cid = lax.axis_index("c")
    sid = lax.axis_index("s")
    tid = cid * NS + sid

    pltpu.sync_copy(src_hbm.at[tid], src_v)
    pltpu.sync_copy(dst_hbm.at[tid], dst_v)
    pltpu.sync_copy(c_hbm, c_v)
    cvec = c_v[...]

    zeros16 = jnp.zeros((16,), jnp.float32)
    base = sid * ROWS_PER_TILE

    @plsc.parallel_loop(0, G, unroll=4)
    def _zero_zbuf(r):
        for j in range(DH // 16):
            zbuf[r, pl.ds(j * 16, 16)] = zeros16

    def _phase1(asrc_v, adst_v, s_v):
        pltpu.sync_copy(asrc_hbm, asrc_v)
        pltpu.sync_copy(adst_hbm, adst_v)

        @plsc.parallel_loop(0, N_PAD // 16, unroll=4)
        def _zero_s(i):
            s_v[pl.ds(i * 16, 16)] = zeros16

        @plsc.parallel_loop(0, NG, unroll=2)
        def _p1(g):
            for k in range(8):
                sl = pl.ds(k * 16, 16)
                i_s = src_v[g, sl]
                i_d = dst_v[g, sl]
                a = (plsc.load_gather(asrc_v, [i_s])
                     + plsc.load_gather(adst_v, [i_d]))
                a = jnp.maximum(a, 0.2 * a) - cvec
                e = jnp.exp(a)
                e_v[g, sl] = e
                plsc.addupdate_scatter(s_v, [i_d], e)
        pltpu.sync_copy(s_v, sparts_hbm.at[tid])

    pl.run_scoped(_phase1,
                  pltpu.VMEM((N_PAD,), jnp.float32),
                  pltpu.VMEM((N_PAD,), jnp.float32),
                  pltpu.VMEM((N_PAD,), jnp.float32))

    def _phase2(gbuf, sbuf):
        gsems = (gsem0, gsem1)
        ssems = (ssem0, ssem1)
        for q in range(D // DH):
            stage = pltpu.make_async_copy(
                hp_hbm.at[pl.ds(base, ROWS_PER_TILE), pl.ds(q * DH, DH)],
                hps.at[pl.ds(base, ROWS_PER_TILE)], gsem0)
            stage.start()
            zs = [pltpu.make_async_copy(zbuf,
                                        acc.at[pl.ds(base + k * G, G)],
                                        ssem0)
                  for k in range(ROWS_PER_TILE // G)]
            for z in zs:
                z.start()
            stage.wait()
            for z in zs:
                z.wait()
            plsc.subcore_barrier()

            pltpu.async_copy(hps.at[src_v.at[0]], gbuf.at[0], gsem0)
            pltpu.async_copy(hps.at[src_v.at[1]], gbuf.at[1], gsem1)

            def _pair(t, _):
                for p in range(2):
                    g = 2 * t + p
                    pltpu.make_async_copy(hps.at[src_v.at[g]], gbuf.at[p],
                                          gsems[p]).wait()

                    @pl.when(t > 0)
                    def _wait_scatter():
                        pltpu.make_async_copy(sbuf.at[p],
                                              acc.at[dst_v.at[g]],
                                              ssems[p]).wait()

                    @plsc.parallel_loop(0, G // 16, unroll=4)
                    def _scale(kk):
                        e16 = e_v[g, pl.ds(kk * 16, 16)]
                        for r0 in range(16):
                            cr = e16[r0]
                            r = kk * 16 + r0
                            for j in range(DH // 16):
                                sl = pl.ds(j * 16, 16)
                                sbuf[p, r, sl] = gbuf[p, r, sl] * cr

                    @pl.when(g + 2 < NG)
                    def _next_gather():
                        pltpu.async_copy(hps.at[src_v.at[g + 2]], gbuf.at[p],
                                         gsems[p])

                    pltpu.async_copy(sbuf.at[p], acc.at[dst_v.at[g]],
                                     ssems[p], add=True)
                return 0

            lax.fori_loop(0, NG // 2, _pair, 0)
            for p in range(2):
                pltpu.make_async_copy(sbuf.at[p], acc.at[dst_v.at[0]],
                                      ssems[p]).wait()
            plsc.subcore_barrier()
            pltpu.sync_copy(acc.at[pl.ds(base, ROWS_PER_TILE)],
                            outparts_hbm.at[cid, pl.ds(base, ROWS_PER_TILE),
                                            pl.ds(q * DH, DH)])

    pl.run_scoped(_phase2,
                  pltpu.VMEM((2, G, DH), jnp.float32),
                  pltpu.VMEM((2, G, DH), jnp.float32))


def _sc_edge_pass(hp, asrc, adst, src3, dst3, cvec):
    mesh = plsc.VectorSubcoreMesh(core_axis_name="c", subcore_axis_name="s",
                                  num_cores=NC, num_subcores=NS)
    f = pl.kernel(
        _sc_body,
        out_type=[
            jax.ShapeDtypeStruct((NC, N_PAD, D), jnp.float32),
            jax.ShapeDtypeStruct((TILES, N_PAD), jnp.float32),
        ],
        mesh=mesh,
        compiler_params=pltpu.CompilerParams(needs_layout_passes=False,
                                             use_tc_tiling_on_sc=False),
        scratch_types=[
            pltpu.VMEM((NG, G), jnp.int32),
            pltpu.VMEM((NG, G), jnp.int32),
            pltpu.VMEM((NG, G), jnp.float32),
            pltpu.VMEM((16,), jnp.float32),
            pltpu.VMEM((G, DH), jnp.float32),
            pltpu.VMEM_SHARED((N_PAD, DH), jnp.float32),
            pltpu.VMEM_SHARED((N_PAD, DH), jnp.float32),
            pltpu.SemaphoreType.DMA,
            pltpu.SemaphoreType.DMA,
            pltpu.SemaphoreType.DMA,
            pltpu.SemaphoreType.DMA,
        ],
    )
    return f(hp, asrc, adst, src3, dst3, cvec)


def _project_body(h_ref, w_ref, as_ref, ad_ref, slab_ref, yacc_ref,
                  hp_ref, aa_ref, c_ref, yout_ref):
    h = h_ref[...]
    w = w_ref[...]
    hp_ref[...] = jnp.dot(h, w, preferred_element_type=jnp.float32)
    v1 = lax.dot_general(w, as_ref[...], (((1,), (1,)), ((), ())))
    v2 = lax.dot_general(w, ad_ref[...], (((1,), (1,)), ((), ())))
    a_s = jnp.dot(h, v1, preferred_element_type=jnp.float32)
    a_d = jnp.dot(h, v2, preferred_element_type=jnp.float32)
    aa_ref[...] = jnp.concatenate([a_s, a_d], axis=1)
    c = jnp.maximum(jnp.max(a_s) + jnp.max(a_d), 0.0)
    c_ref[...] = jnp.full((1, 128), c, jnp.float32)
    yout_ref[...] = yacc_ref[...] + jnp.dot(
        h, slab_ref[...], preferred_element_type=jnp.float32)


def _project(h, w, a_s, a_d, slab, yacc):
    return pl.pallas_call(
        _project_body,
        out_shape=[
            jax.ShapeDtypeStruct((N_PAD, D), jnp.float32),
            jax.ShapeDtypeStruct((N_PAD, 2), jnp.float32),
            jax.ShapeDtypeStruct((1, 128), jnp.float32),
            jax.ShapeDtypeStruct((N_PAD, D), jnp.float32),
        ],
    )(h, w, a_s.reshape(1, D), a_d.reshape(1, D), slab, yacc)


def _norm_h(p0, p1, sparts):
    agg = p0 + p1
    ones = jnp.ones((TILES, 1), jnp.float32)
    s = lax.dot_general(sparts, ones, (((0,), (0,)), ((), ())))
    y = agg * (1.0 / (s + 1e-16))
    mask = lax.broadcasted_iota(jnp.int32, (N_PAD, 1), 0) < N
    y = jnp.where(mask, y, 0.0)
    mu = jnp.sum(y, axis=0, keepdims=True) * (1.0 / N)
    yc = jnp.where(mask, y - mu, 0.0)
    msq = jnp.sum(yc * yc) * (1.0 / N)
    x = yc * lax.rsqrt(1e-5 + msq)
    return x * 0.5 * (1.0 + lax.erf(x * (2.0 ** -0.5)))


def _mid_body(p0_ref, p1_ref, sparts_ref, w_ref, as_ref, ad_ref, slab_ref,
              yacc_ref, hp_ref, aa_ref, c_ref, yout_ref):
    h = _norm_h(p0_ref[...], p1_ref[...], sparts_ref[...])
    w = w_ref[...]
    hp_ref[...] = jnp.dot(h, w, preferred_element_type=jnp.float32)
    v1 = lax.dot_general(w, as_ref[...], (((1,), (1,)), ((), ())))
    v2 = lax.dot_general(w, ad_ref[...], (((1,), (1,)), ((), ())))
    a_s = jnp.dot(h, v1, preferred_element_type=jnp.float32)
    a_d = jnp.dot(h, v2, preferred_element_type=jnp.float32)
    aa_ref[...] = jnp.concatenate([a_s, a_d], axis=1)
    c = jnp.maximum(jnp.max(a_s) + jnp.max(a_d), 0.0)
    c_ref[...] = jnp.full((1, 128), c, jnp.float32)
    yout_ref[...] = yacc_ref[...] + jnp.dot(
        h, slab_ref[...], preferred_element_type=jnp.float32)


def _mid(outparts, sparts, w, a_s, a_d, slab, yacc):
    return pl.pallas_call(
        _mid_body,
        out_shape=[
            jax.ShapeDtypeStruct((N_PAD, D), jnp.float32),
            jax.ShapeDtypeStruct((N_PAD, 2), jnp.float32),
            jax.ShapeDtypeStruct((1, 128), jnp.float32),
            jax.ShapeDtypeStruct((N_PAD, D), jnp.float32),
        ],
    )(outparts[0], outparts[1], sparts, w, a_s.reshape(1, D),
      a_d.reshape(1, D), slab, yacc)


def _final_body(p0_ref, p1_ref, sparts_ref, slab_ref, b_ref, yacc_ref, y_ref):
    h = _norm_h(p0_ref[...], p1_ref[...], sparts_ref[...])
    y_ref[...] = (yacc_ref[...]
                  + jnp.dot(h, slab_ref[...],
                            preferred_element_type=jnp.float32)
                  + b_ref[...])


def _final(outparts, sparts, slab, lin_b, yacc):
    return pl.pallas_call(
        _final_body,
        out_shape=jax.ShapeDtypeStruct((N_PAD, D), jnp.float32),
    )(outparts[0], outparts[1], sparts, slab, lin_b.reshape(1, D), yacc)


def kernel(x, edge_index, Ws, att_src, att_dst, biases, lin_W, lin_b):
    loop = jnp.arange(N, dtype=edge_index.dtype)
    pad = jnp.full((E_PAD - E_TOT,), N, dtype=edge_index.dtype)
    src = jnp.concatenate([edge_index[0], loop, pad]).astype(jnp.int32)
    dst = jnp.concatenate([edge_index[1], loop, pad]).astype(jnp.int32)
    src3 = src.reshape(TILES, NG, G)
    dst3 = dst.reshape(TILES, NG, G)

    x_pad = jnp.pad(x, ((0, N_PAD - N), (0, 0)))
    yacc = jnp.zeros((N_PAD, D), jnp.float32)
    zero_slab = jnp.zeros((D, D), jnp.float32)

    hp, aa, cmat, yacc = _project(x_pad, Ws[0], att_src[0], att_dst[0],
                                  zero_slab, yacc)
    for l in range(L):
        asrc = aa[:, 0].reshape(N_PAD)
        adst = aa[:, 1].reshape(N_PAD)
        cvec = cmat[0, :16].reshape(16)
        outparts, sparts = _sc_edge_pass(hp, asrc, adst, src3, dst3, cvec)
        if l < L - 1:
            hp, aa, cmat, yacc = _mid(outparts, sparts, Ws[l + 1],
                                      att_src[l + 1], att_dst[l + 1],
                                      lin_W[l * D:(l + 1) * D], yacc)
    y = _final(outparts, sparts, lin_W[(L - 1) * D:], lin_b, yacc)
    return y[:N]
